# baseline (speedup 1.0000x reference)
import functools

import jax
import jax.numpy as jnp
from jax import lax
from jax.experimental import pallas as pl
from jax.experimental.pallas import tpu as pltpu
from jax.experimental.pallas import tpu_sc as plsc

N = 10000
E = 320000
K = 64
NC, NS, L = 2, 16, 16
NW = NC * NS
EPW = E // NW
N_PAD = NW * 320
NPW = N_PAD // NW
RPS = N_PAD // NS
CH = 80
KPB = 25
NB3 = E // (KPB * CH)
NBB = NB3 // NW
CH2 = 2000
F = 128

_mesh = plsc.VectorSubcoreMesh(
    core_axis_name="c", subcore_axis_name="s", num_cores=NC, num_subcores=NS)
_sc_params = pltpu.CompilerParams(needs_layout_passes=False)


def _rsqrt16(d):
    bits = plsc.bitcast(d, jnp.int32)
    y = plsc.bitcast(jnp.int32(0x5F3759DF) - (bits >> 1), jnp.float32)
    h = d * jnp.float32(-0.5)
    y = y * (jnp.float32(1.5) + h * y * y)
    y = y * (jnp.float32(1.5) + h * y * y)
    y = y * (jnp.float32(1.5) + h * y * y)
    return y


@functools.partial(
    pl.kernel,
    out_type=(jax.ShapeDtypeStruct((NB3, KPB, CH), jnp.float32),
              jax.ShapeDtypeStruct((N_PAD,), jnp.float32)),
    mesh=_mesh,
    scratch_types=[
        pltpu.VMEM_SHARED((N_PAD,), jnp.float32),
        pltpu.VMEM((N_PAD,), jnp.float32),
        pltpu.VMEM((N_PAD // NS,), jnp.float32),
        pltpu.VMEM((KPB, CH), jnp.int32),
        pltpu.VMEM((KPB, CH), jnp.float32),
        pltpu.VMEM((CH2,), jnp.int32),
        pltpu.VMEM((CH2,), jnp.float32),
        pltpu.VMEM((KPB, CH), jnp.float32),
        pltpu.SemaphoreType.DMA,
    ],
    compiler_params=_sc_params,
)
def _norm_kernel(row_h, ew_h, col2_h, ew2_h, norm2_h, dinv_h,
                 deg_sh, dloc, onesb, cbufD, ebufD, rbufN, ebufN, nbufN, semD):
    c = lax.axis_index("c")
    s = lax.axis_index("s")
    wid = s * NC + c

    spn = N_PAD // NS

    def ones_fill(i, carry):
        onesb[pl.ds(i * L, L)] = jnp.full((L,), 1.0, jnp.float32)
        return carry
    lax.fori_loop(0, spn // L, ones_fill, 0)
    pltpu.sync_copy(onesb, deg_sh.at[pl.ds(s * spn, spn)])
    plsc.subcore_barrier()

    bps = NB3 // NS

    def deg_block(b, carry):
        blk = s * bps + b
        pltpu.sync_copy(col2_h.at[blk], cbufD)
        pltpu.sync_copy(ew2_h.at[blk], ebufD)
        hs = [pltpu.async_copy(ebufD.at[k], deg_sh.at[cbufD.at[k]],
                               semD, add=True) for k in range(KPB)]
        for h in hs:
            h.wait()
        return carry
    lax.fori_loop(0, bps, deg_block, 0)
    plsc.subcore_barrier()

    pltpu.sync_copy(deg_sh, dloc)

    def rsq(i, carry):
        dloc[pl.ds(i * L, L)] = _rsqrt16(dloc[pl.ds(i * L, L)])
        return carry
    lax.fori_loop(0, N_PAD // L, rsq, 0)
    pltpu.sync_copy(dloc.at[pl.ds(wid * NPW, NPW)],
                    dinv_h.at[pl.ds(wid * NPW, NPW)])

    def norm_chunk(i, carry):
        off = wid * EPW + i * CH2
        pltpu.sync_copy(row_h.at[pl.ds(off, CH2)], rbufN)
        pltpu.sync_copy(ew_h.at[pl.ds(off, CH2)], ebufN)

        def inner(j, carry2):
            r = rbufN[pl.ds(j * L, L)]
            e = ebufN[pl.ds(j * L, L)]
            a = plsc.load_gather(dloc, [r])
            kk = j // (CH // L)
            jj = j % (CH // L)
            nbufN[kk, pl.ds(jj * L, L)] = a * e
            return carry2
        lax.fori_loop(0, CH2 // L, inner, 0)
        pltpu.sync_copy(nbufN, norm2_h.at[wid * NBB + i])
        return carry
    lax.fori_loop(0, EPW // CH2, norm_chunk, 0)


@functools.partial(
    pl.kernel,
    out_type=jax.ShapeDtypeStruct((NC, N_PAD, F), jnp.float32),
    mesh=_mesh,
    scratch_types=[
        pltpu.VMEM_SHARED((N_PAD, F), jnp.float32),
        pltpu.VMEM((3, CH, F), jnp.float32),
        pltpu.VMEM((KPB, CH), jnp.int32),
        pltpu.VMEM((KPB, CH), jnp.int32),
        pltpu.VMEM((KPB, CH), jnp.float32),
        pltpu.SemaphoreType.DMA,
        pltpu.SemaphoreType.DMA,
        pltpu.SemaphoreType.DMA,
        pltpu.SemaphoreType.DMA,
        pltpu.SemaphoreType.DMA,
        pltpu.SemaphoreType.DMA,
    ],
    compiler_params=_sc_params,
)
def _spmm_kernel(y_h, r2_h, c2_h, n2_h, out_h,
                 acc_sh, gbuf, rbuf, cbuf, nbuf, g0, g1, g2, s0, s1, s2):
    gsems = [g0, g1, g2]
    ssems = [s0, s1, s2]
    c = lax.axis_index("c")
    s = lax.axis_index("s")
    wid = s * NC + c
    gb = [gbuf.at[r] for r in range(3)]

    z = gb[0]

    def zrow(i, carry):
        for k in range(F // L):
            z[i, pl.ds(k * L, L)] = jnp.zeros((L,), jnp.float32)
        return carry
    lax.fori_loop(0, CH, zrow, 0)
    for t in range(RPS // CH):
        pltpu.sync_copy(z, acc_sh.at[pl.ds(s * RPS + t * CH, CH)])
    plsc.subcore_barrier()

    def scale_block(k, sl):
        g = gb[sl]
        kk = jnp.full((L,), k, jnp.int32)

        def srow(j, carry2):
            for u in range(8):
                jj = j * 8 + u
                nv = plsc.load_gather(
                    nbuf, [kk, jnp.full((L,), 0, jnp.int32) + jj])
                for t in range(F // L):
                    g[jj, pl.ds(t * L, L)] = g[jj, pl.ds(t * L, L)] * nv
            return carry2
        lax.fori_loop(0, CH // 8, srow, 0)

    def block(b, carry):
        blk = wid * NBB + b
        pltpu.sync_copy(r2_h.at[blk], rbuf)
        pltpu.sync_copy(c2_h.at[blk], cbuf)
        pltpu.sync_copy(n2_h.at[blk], nbuf)
        gh = [None] * 3
        sh = [None] * 3
        gh[0] = pltpu.async_copy(y_h.at[rbuf.at[0]], gb[0], gsems[0])
        gh[1] = pltpu.async_copy(y_h.at[rbuf.at[1]], gb[1], gsems[1])
        for k in range(KPB):
            sl = k % 3
            gh[sl].wait()
            scale_block(k, sl)
            sh[sl] = pltpu.async_copy(gb[sl], acc_sh.at[cbuf.at[k]],
                                      ssems[sl], add=True)
            if k >= 1:
                sh[(k - 1) % 3].wait()
            if k + 2 < KPB:
                nsl = (k + 2) % 3
                gh[nsl] = pltpu.async_copy(y_h.at[rbuf.at[k + 2]], gb[nsl],
                                           gsems[nsl])
        sh[(KPB - 1) % 3].wait()
        return carry
    lax.fori_loop(0, NBB, block, 0)
    plsc.subcore_barrier()

    rows = pl.ds(s * RPS, RPS)
    pltpu.sync_copy(acc_sh.at[rows], out_h.at[c, rows])


BM = 1000


def _mm_body(x_ref, w_ref, o_ref):
    o_ref[...] = jnp.dot(x_ref[...], w_ref[...],
                         preferred_element_type=jnp.float32)


def _mm(x, W):
    M, Kin = x.shape
    Kout = W.shape[1]
    return pl.pallas_call(
        _mm_body,
        grid=(M // BM,),
        in_specs=[pl.BlockSpec((BM, Kin), lambda i: (i, 0)),
                  pl.BlockSpec((Kin, Kout), lambda i: (0, 0))],
        out_specs=pl.BlockSpec((BM, Kout), lambda i: (i, 0)),
        out_shape=jax.ShapeDtypeStruct((M, Kout), jnp.float32),
    )(x, W)


def _ep1_body(ap_ref, xw_ref, dv_ref, b_ref, w_ref, o_ref):
    dv = dv_ref[...]
    h = dv * (ap_ref[0] + ap_ref[1]) + (dv * dv) * xw_ref[...] + b_ref[...]
    h = jnp.maximum(h, 0.0)
    o_ref[...] = jnp.dot(h, w_ref[...], preferred_element_type=jnp.float32)


def _ep1(ap, xw, dv, b, W):
    M, Fin = xw.shape
    Kout = W.shape[1]
    return pl.pallas_call(
        _ep1_body,
        grid=(M // BM,),
        in_specs=[pl.BlockSpec((NC, BM, Fin), lambda i: (0, i, 0)),
                  pl.BlockSpec((BM, Fin), lambda i: (i, 0)),
                  pl.BlockSpec((BM, 1), lambda i: (i, 0)),
                  pl.BlockSpec((1, Fin), lambda i: (0, 0)),
                  pl.BlockSpec((Fin, Kout), lambda i: (0, 0))],
        out_specs=pl.BlockSpec((BM, Kout), lambda i: (i, 0)),
        out_shape=jax.ShapeDtypeStruct((M, Kout), jnp.float32),
    )(ap, xw, dv, b, W)


def _tail_body(ap_ref, xw_ref, dv_ref, b_ref, eps_ref,
               mu_ref, std_ref, z_ref):
    dv = dv_ref[...]
    h = dv * (ap_ref[0] + ap_ref[1]) + (dv * dv) * xw_ref[...] + b_ref[...]
    mu = h[:, :K]
    t = h[:, K:] - 5.0
    sp = jnp.maximum(t, 0.0) + jnp.log1p(jnp.exp(-jnp.abs(t)))
    mu_ref[...] = mu
    std_ref[...] = sp
    z_ref[...] = mu + eps_ref[...] * sp


def _tail(ap, xw, dv, b, eps):
    M, Fin = xw.shape
    out = jax.ShapeDtypeStruct((M, K), jnp.float32)
    return pl.pallas_call(
        _tail_body,
        grid=(M // BM,),
        in_specs=[pl.BlockSpec((NC, BM, Fin), lambda i: (0, i, 0)),
                  pl.BlockSpec((BM, Fin), lambda i: (i, 0)),
                  pl.BlockSpec((BM, 1), lambda i: (i, 0)),
                  pl.BlockSpec((1, Fin), lambda i: (0, 0)),
                  pl.BlockSpec((BM, K), lambda i: (i, 0))],
        out_specs=[pl.BlockSpec((BM, K), lambda i: (i, 0))] * 3,
        out_shape=(out, out, out),
    )(ap, xw, dv, b, eps)


def kernel(x, edge_index, edge_weight, W1, b1, W2, b2):
    row = edge_index[0]
    col = edge_index[1]
    row2 = row.reshape(NB3, KPB, CH)
    col2 = col.reshape(NB3, KPB, CH)
    ew2 = edge_weight.reshape(NB3, KPB, CH)

    norm2, dinv = _norm_kernel(row, edge_weight, col2, ew2)
    dv = dinv[:N].reshape(N, 1)

    xw1 = _mm(x, W1)
    ap1 = _spmm_kernel(xw1, row2, col2, norm2)
    xw2 = _ep1(ap1, xw1, dv, b1.reshape(1, -1), W2)

    ap2 = _spmm_kernel(xw2, row2, col2, norm2)
    eps = jax.random.normal(jax.random.key(42), (N, K), jnp.float32)
    mu, std, z = _tail(ap2, xw2, dv, b2.reshape(1, -1), eps)
    return (mu, std, z)


# speedup vs baseline: 1.0277x; 1.0277x over previous
import functools

import jax
import jax.numpy as jnp
from jax import lax
from jax.experimental import pallas as pl
from jax.experimental.pallas import tpu as pltpu
from jax.experimental.pallas import tpu_sc as plsc

N = 10000
E = 320000
K = 64
NC, NS, L = 2, 16, 16
NW = NC * NS
EPW = E // NW
N_PAD = NW * 320
NPW = N_PAD // NW
RPS = N_PAD // NS
CH = 80
KPB = 25
NB3 = E // (KPB * CH)
NBB = NB3 // NW
CH2 = 2000
F = 128

_mesh = plsc.VectorSubcoreMesh(
    core_axis_name="c", subcore_axis_name="s", num_cores=NC, num_subcores=NS)
_sc_params = pltpu.CompilerParams(needs_layout_passes=False)


def _rsqrt16(d):
    bits = plsc.bitcast(d, jnp.int32)
    y = plsc.bitcast(jnp.int32(0x5F3759DF) - (bits >> 1), jnp.float32)
    h = d * jnp.float32(-0.5)
    y = y * (jnp.float32(1.5) + h * y * y)
    y = y * (jnp.float32(1.5) + h * y * y)
    y = y * (jnp.float32(1.5) + h * y * y)
    return y


@functools.partial(
    pl.kernel,
    out_type=(jax.ShapeDtypeStruct((NB3, KPB, CH), jnp.float32),
              jax.ShapeDtypeStruct((N_PAD,), jnp.float32)),
    mesh=_mesh,
    scratch_types=[
        pltpu.VMEM_SHARED((N_PAD,), jnp.float32),
        pltpu.VMEM((N_PAD,), jnp.float32),
        pltpu.VMEM((N_PAD // NS,), jnp.float32),
        pltpu.VMEM((2, KPB, CH), jnp.int32),
        pltpu.VMEM((2, KPB, CH), jnp.float32),
        pltpu.VMEM((CH2,), jnp.int32),
        pltpu.VMEM((CH2,), jnp.float32),
        pltpu.VMEM((KPB, CH), jnp.float32),
        pltpu.SemaphoreType.DMA,
    ],
    compiler_params=_sc_params,
)
def _norm_kernel(row_h, ew_h, col2_h, ew2_h, norm2_h, dinv_h,
                 deg_sh, dloc, onesb, cbufD, ebufD, rbufN, ebufN, nbufN, semD):
    c = lax.axis_index("c")
    s = lax.axis_index("s")
    wid = s * NC + c

    spn = N_PAD // NS

    def ones_fill(i, carry):
        onesb[pl.ds(i * L, L)] = jnp.full((L,), 1.0, jnp.float32)
        return carry
    lax.fori_loop(0, spn // L, ones_fill, 0)
    pltpu.sync_copy(onesb, deg_sh.at[pl.ds(s * spn, spn)])
    plsc.subcore_barrier()

    bps = NB3 // NS
    cbs = [cbufD.at[r] for r in range(2)]
    ebs = [ebufD.at[r] for r in range(2)]

    def stage(blk, r):
        pltpu.sync_copy(col2_h.at[blk], cbs[r])
        pltpu.sync_copy(ew2_h.at[blk], ebs[r])

    def fire(r):
        return [pltpu.async_copy(ebs[r].at[k], deg_sh.at[cbs[r].at[k]],
                                 semD, add=True) for k in range(KPB)]

    stage(s * bps, 0)

    def deg_pair(p, carry):
        for r in range(2):
            b = p * 2 + r
            hs = fire(r)

            @pl.when(b + 1 < bps)
            def _():
                stage(s * bps + b + 1, (r + 1) % 2)
            for h in hs:
                h.wait()
        return carry
    lax.fori_loop(0, bps // 2, deg_pair, 0)
    plsc.subcore_barrier()

    pltpu.sync_copy(deg_sh, dloc)

    def rsq(i, carry):
        dloc[pl.ds(i * L, L)] = _rsqrt16(dloc[pl.ds(i * L, L)])
        return carry
    lax.fori_loop(0, N_PAD // L, rsq, 0)
    pltpu.sync_copy(dloc.at[pl.ds(wid * NPW, NPW)],
                    dinv_h.at[pl.ds(wid * NPW, NPW)])

    def norm_chunk(i, carry):
        off = wid * EPW + i * CH2
        pltpu.sync_copy(row_h.at[pl.ds(off, CH2)], rbufN)
        pltpu.sync_copy(ew_h.at[pl.ds(off, CH2)], ebufN)

        def inner(j, carry2):
            r = rbufN[pl.ds(j * L, L)]
            e = ebufN[pl.ds(j * L, L)]
            a = plsc.load_gather(dloc, [r])
            kk = j // (CH // L)
            jj = j % (CH // L)
            nbufN[kk, pl.ds(jj * L, L)] = a * e
            return carry2
        lax.fori_loop(0, CH2 // L, inner, 0)
        pltpu.sync_copy(nbufN, norm2_h.at[wid * NBB + i])
        return carry
    lax.fori_loop(0, EPW // CH2, norm_chunk, 0)


@functools.partial(
    pl.kernel,
    out_type=jax.ShapeDtypeStruct((NC, N_PAD, F), jnp.float32),
    mesh=_mesh,
    scratch_types=[
        pltpu.VMEM_SHARED((N_PAD, F), jnp.float32),
        pltpu.VMEM((3, CH, F), jnp.float32),
        pltpu.VMEM((KPB, CH), jnp.int32),
        pltpu.VMEM((KPB, CH), jnp.int32),
        pltpu.VMEM((KPB, CH), jnp.float32),
        pltpu.SemaphoreType.DMA,
        pltpu.SemaphoreType.DMA,
        pltpu.SemaphoreType.DMA,
        pltpu.SemaphoreType.DMA,
        pltpu.SemaphoreType.DMA,
        pltpu.SemaphoreType.DMA,
    ],
    compiler_params=_sc_params,
)
def _spmm_kernel(y_h, r2_h, c2_h, n2_h, out_h,
                 acc_sh, gbuf, rbuf, cbuf, nbuf, g0, g1, g2, s0, s1, s2):
    gsems = [g0, g1, g2]
    ssems = [s0, s1, s2]
    c = lax.axis_index("c")
    s = lax.axis_index("s")
    wid = s * NC + c
    gb = [gbuf.at[r] for r in range(3)]

    z = gb[0]

    def zrow(i, carry):
        for k in range(F // L):
            z[i, pl.ds(k * L, L)] = jnp.zeros((L,), jnp.float32)
        return carry
    lax.fori_loop(0, CH, zrow, 0)
    for t in range(RPS // CH):
        pltpu.sync_copy(z, acc_sh.at[pl.ds(s * RPS + t * CH, CH)])
    plsc.subcore_barrier()

    def scale_block(k, sl):
        g = gb[sl]
        kk = jnp.full((L,), k, jnp.int32)

        def srow(j, carry2):
            for u in range(4):
                jj = j * 4 + u
                nv = plsc.load_gather(
                    nbuf, [kk, jnp.full((L,), 0, jnp.int32) + jj])
                for t in range(F // L):
                    g[jj, pl.ds(t * L, L)] = g[jj, pl.ds(t * L, L)] * nv
            return carry2
        lax.fori_loop(0, CH // 4, srow, 0)

    def block(b, carry):
        blk = wid * NBB + b
        pltpu.sync_copy(r2_h.at[blk], rbuf)
        pltpu.sync_copy(c2_h.at[blk], cbuf)
        pltpu.sync_copy(n2_h.at[blk], nbuf)
        gh = [None] * 3
        sh = [None] * 3
        gh[0] = pltpu.async_copy(y_h.at[rbuf.at[0]], gb[0], gsems[0])
        gh[1] = pltpu.async_copy(y_h.at[rbuf.at[1]], gb[1], gsems[1])
        for k in range(KPB):
            sl = k % 3
            gh[sl].wait()
            scale_block(k, sl)
            sh[sl] = pltpu.async_copy(gb[sl], acc_sh.at[cbuf.at[k]],
                                      ssems[sl], add=True)
            if k >= 1:
                sh[(k - 1) % 3].wait()
            if k + 2 < KPB:
                nsl = (k + 2) % 3
                gh[nsl] = pltpu.async_copy(y_h.at[rbuf.at[k + 2]], gb[nsl],
                                           gsems[nsl])
        sh[(KPB - 1) % 3].wait()
        return carry
    lax.fori_loop(0, NBB, block, 0)
    plsc.subcore_barrier()

    rows = pl.ds(s * RPS, RPS)
    pltpu.sync_copy(acc_sh.at[rows], out_h.at[c, rows])


BM = 1000


def _mm_body(x_ref, w_ref, o_ref):
    o_ref[...] = jnp.dot(x_ref[...], w_ref[...],
                         preferred_element_type=jnp.float32)


def _mm(x, W):
    M, Kin = x.shape
    Kout = W.shape[1]
    return pl.pallas_call(
        _mm_body,
        grid=(M // BM,),
        in_specs=[pl.BlockSpec((BM, Kin), lambda i: (i, 0)),
                  pl.BlockSpec((Kin, Kout), lambda i: (0, 0))],
        out_specs=pl.BlockSpec((BM, Kout), lambda i: (i, 0)),
        out_shape=jax.ShapeDtypeStruct((M, Kout), jnp.float32),
    )(x, W)


def _ep1_body(ap_ref, xw_ref, dv_ref, b_ref, w_ref, o_ref):
    dv = dv_ref[...]
    h = dv * (ap_ref[0] + ap_ref[1]) + (dv * dv) * xw_ref[...] + b_ref[...]
    h = jnp.maximum(h, 0.0)
    o_ref[...] = jnp.dot(h, w_ref[...], preferred_element_type=jnp.float32)


def _ep1(ap, xw, dv, b, W):
    M, Fin = xw.shape
    Kout = W.shape[1]
    return pl.pallas_call(
        _ep1_body,
        grid=(M // BM,),
        in_specs=[pl.BlockSpec((NC, BM, Fin), lambda i: (0, i, 0)),
                  pl.BlockSpec((BM, Fin), lambda i: (i, 0)),
                  pl.BlockSpec((BM, 1), lambda i: (i, 0)),
                  pl.BlockSpec((1, Fin), lambda i: (0, 0)),
                  pl.BlockSpec((Fin, Kout), lambda i: (0, 0))],
        out_specs=pl.BlockSpec((BM, Kout), lambda i: (i, 0)),
        out_shape=jax.ShapeDtypeStruct((M, Kout), jnp.float32),
    )(ap, xw, dv, b, W)


def _tail_body(ap_ref, xw_ref, dv_ref, b_ref, eps_ref,
               mu_ref, std_ref, z_ref):
    dv = dv_ref[...]
    h = dv * (ap_ref[0] + ap_ref[1]) + (dv * dv) * xw_ref[...] + b_ref[...]
    mu = h[:, :K]
    t = h[:, K:] - 5.0
    sp = jnp.maximum(t, 0.0) + jnp.log1p(jnp.exp(-jnp.abs(t)))
    mu_ref[...] = mu
    std_ref[...] = sp
    z_ref[...] = mu + eps_ref[...] * sp


def _tail(ap, xw, dv, b, eps):
    M, Fin = xw.shape
    out = jax.ShapeDtypeStruct((M, K), jnp.float32)
    return pl.pallas_call(
        _tail_body,
        grid=(M // BM,),
        in_specs=[pl.BlockSpec((NC, BM, Fin), lambda i: (0, i, 0)),
                  pl.BlockSpec((BM, Fin), lambda i: (i, 0)),
                  pl.BlockSpec((BM, 1), lambda i: (i, 0)),
                  pl.BlockSpec((1, Fin), lambda i: (0, 0)),
                  pl.BlockSpec((BM, K), lambda i: (i, 0))],
        out_specs=[pl.BlockSpec((BM, K), lambda i: (i, 0))] * 3,
        out_shape=(out, out, out),
    )(ap, xw, dv, b, eps)


def kernel(x, edge_index, edge_weight, W1, b1, W2, b2):
    row = edge_index[0]
    col = edge_index[1]
    row2 = row.reshape(NB3, KPB, CH)
    col2 = col.reshape(NB3, KPB, CH)
    ew2 = edge_weight.reshape(NB3, KPB, CH)

    norm2, dinv = _norm_kernel(row, edge_weight, col2, ew2)
    dv = dinv[:N].reshape(N, 1)

    xw1 = _mm(x, W1)
    ap1 = _spmm_kernel(xw1, row2, col2, norm2)
    xw2 = _ep1(ap1, xw1, dv, b1.reshape(1, -1), W2)

    ap2 = _spmm_kernel(xw2, row2, col2, norm2)
    eps = jax.random.normal(jax.random.key(42), (N, K), jnp.float32)
    mu, std, z = _tail(ap2, xw2, dv, b2.reshape(1, -1), eps)
    return (mu, std, z)


# speedup vs baseline: 1.1143x; 1.0843x over previous
import functools

import jax
import jax.numpy as jnp
from jax import lax
from jax.experimental import pallas as pl
from jax.experimental.pallas import tpu as pltpu
from jax.experimental.pallas import tpu_sc as plsc

N = 10000
E = 320000
K = 64
NC, NS, L = 2, 16, 16
NW = NC * NS
EPW = E // NW
N_PAD = NW * 320
NPW = N_PAD // NW
RPS = N_PAD // NS
CH = 80
KPB = 25
NB3 = E // (KPB * CH)
NBB = NB3 // NW
CH2 = 2000
F = 128

_mesh = plsc.VectorSubcoreMesh(
    core_axis_name="c", subcore_axis_name="s", num_cores=NC, num_subcores=NS)
_sc_params = pltpu.CompilerParams(needs_layout_passes=False)


def _rsqrt16(d):
    bits = plsc.bitcast(d, jnp.int32)
    y = plsc.bitcast(jnp.int32(0x5F3759DF) - (bits >> 1), jnp.float32)
    h = d * jnp.float32(-0.5)
    y = y * (jnp.float32(1.5) + h * y * y)
    y = y * (jnp.float32(1.5) + h * y * y)
    y = y * (jnp.float32(1.5) + h * y * y)
    return y


@functools.partial(
    pl.kernel,
    out_type=(jax.ShapeDtypeStruct((NB3, KPB, CH), jnp.float32),
              jax.ShapeDtypeStruct((N_PAD,), jnp.float32)),
    mesh=_mesh,
    scratch_types=[
        pltpu.VMEM_SHARED((N_PAD,), jnp.float32),
        pltpu.VMEM((N_PAD,), jnp.float32),
        pltpu.VMEM((N_PAD // NS,), jnp.float32),
        pltpu.VMEM((2, KPB, CH), jnp.int32),
        pltpu.VMEM((2, KPB, CH), jnp.float32),
        pltpu.VMEM((CH2,), jnp.int32),
        pltpu.VMEM((CH2,), jnp.float32),
        pltpu.VMEM((KPB, CH), jnp.float32),
        pltpu.SemaphoreType.DMA,
    ],
    compiler_params=_sc_params,
)
def _norm_kernel(row_h, ew_h, col2_h, ew2_h, norm2_h, dinv_h,
                 deg_sh, dloc, onesb, cbufD, ebufD, rbufN, ebufN, nbufN, semD):
    c = lax.axis_index("c")
    s = lax.axis_index("s")
    wid = s * NC + c

    spn = N_PAD // NS

    def ones_fill(i, carry):
        onesb[pl.ds(i * L, L)] = jnp.full((L,), 1.0, jnp.float32)
        return carry
    lax.fori_loop(0, spn // L, ones_fill, 0)
    pltpu.sync_copy(onesb, deg_sh.at[pl.ds(s * spn, spn)])
    plsc.subcore_barrier()

    bps = NB3 // NS
    cbs = [cbufD.at[r] for r in range(2)]
    ebs = [ebufD.at[r] for r in range(2)]

    def stage(blk, r):
        pltpu.sync_copy(col2_h.at[blk], cbs[r])
        pltpu.sync_copy(ew2_h.at[blk], ebs[r])

    def fire(r):
        return [pltpu.async_copy(ebs[r].at[k], deg_sh.at[cbs[r].at[k]],
                                 semD, add=True) for k in range(KPB)]

    stage(s * bps, 0)

    def deg_pair(p, carry):
        for r in range(2):
            b = p * 2 + r
            hs = fire(r)

            @pl.when(b + 1 < bps)
            def _():
                stage(s * bps + b + 1, (r + 1) % 2)
            for h in hs:
                h.wait()
        return carry
    lax.fori_loop(0, bps // 2, deg_pair, 0)
    plsc.subcore_barrier()

    pltpu.sync_copy(deg_sh, dloc)

    def rsq(i, carry):
        dloc[pl.ds(i * L, L)] = _rsqrt16(dloc[pl.ds(i * L, L)])
        return carry
    lax.fori_loop(0, N_PAD // L, rsq, 0)
    pltpu.sync_copy(dloc.at[pl.ds(wid * NPW, NPW)],
                    dinv_h.at[pl.ds(wid * NPW, NPW)])

    def norm_chunk(i, carry):
        off = wid * EPW + i * CH2
        pltpu.sync_copy(row_h.at[pl.ds(off, CH2)], rbufN)
        pltpu.sync_copy(ew_h.at[pl.ds(off, CH2)], ebufN)

        def inner(j, carry2):
            r = rbufN[pl.ds(j * L, L)]
            e = ebufN[pl.ds(j * L, L)]
            a = plsc.load_gather(dloc, [r])
            kk = j // (CH // L)
            jj = j % (CH // L)
            nbufN[kk, pl.ds(jj * L, L)] = a * e
            return carry2
        lax.fori_loop(0, CH2 // L, inner, 0)
        pltpu.sync_copy(nbufN, norm2_h.at[wid * NBB + i])
        return carry
    lax.fori_loop(0, EPW // CH2, norm_chunk, 0)


@functools.partial(
    pl.kernel,
    out_type=jax.ShapeDtypeStruct((NC, N_PAD, F), jnp.float32),
    mesh=_mesh,
    scratch_types=[
        pltpu.VMEM_SHARED((N_PAD, F), jnp.float32),
        pltpu.VMEM((3, CH, F), jnp.float32),
        pltpu.VMEM((KPB, CH), jnp.int32),
        pltpu.VMEM((KPB, CH), jnp.int32),
        pltpu.VMEM((KPB, CH), jnp.float32),
        pltpu.SemaphoreType.DMA,
        pltpu.SemaphoreType.DMA,
        pltpu.SemaphoreType.DMA,
        pltpu.SemaphoreType.DMA,
        pltpu.SemaphoreType.DMA,
        pltpu.SemaphoreType.DMA,
    ],
    compiler_params=_sc_params,
)
def _spmm_kernel(y_h, r2_h, c2_h, n2_h, out_h,
                 acc_sh, gbuf, rbuf, cbuf, nbuf, g0, g1, g2, s0, s1, s2):
    gsems = [g0, g1, g2]
    ssems = [s0, s1, s2]
    c = lax.axis_index("c")
    s = lax.axis_index("s")
    wid = s * NC + c
    gb = [gbuf.at[r] for r in range(3)]

    z = gb[0]

    def zrow(i, carry):
        for k in range(F // L):
            z[i, pl.ds(k * L, L)] = jnp.zeros((L,), jnp.float32)
        return carry
    lax.fori_loop(0, CH, zrow, 0)
    for t in range(RPS // CH):
        pltpu.sync_copy(z, acc_sh.at[pl.ds(s * RPS + t * CH, CH)])
    plsc.subcore_barrier()

    def scale_block(k, sl):
        g = gb[sl]
        kk = jnp.full((L,), k, jnp.int32)

        @plsc.parallel_loop(0, CH, unroll=4)
        def _(jj):
            nv = plsc.load_gather(
                nbuf, [kk, jnp.full((L,), 0, jnp.int32) + jj])
            for t in range(F // L):
                g[jj, pl.ds(t * L, L)] = g[jj, pl.ds(t * L, L)] * nv

    def block(b, carry):
        blk = wid * NBB + b
        pltpu.sync_copy(r2_h.at[blk], rbuf)
        pltpu.sync_copy(c2_h.at[blk], cbuf)
        pltpu.sync_copy(n2_h.at[blk], nbuf)
        gh = [None] * 3
        sh = [None] * 3
        gh[0] = pltpu.async_copy(y_h.at[rbuf.at[0]], gb[0], gsems[0])
        gh[1] = pltpu.async_copy(y_h.at[rbuf.at[1]], gb[1], gsems[1])
        for k in range(KPB):
            sl = k % 3
            gh[sl].wait()
            scale_block(k, sl)
            sh[sl] = pltpu.async_copy(gb[sl], acc_sh.at[cbuf.at[k]],
                                      ssems[sl], add=True)
            if k >= 1:
                sh[(k - 1) % 3].wait()
            if k + 2 < KPB:
                nsl = (k + 2) % 3
                gh[nsl] = pltpu.async_copy(y_h.at[rbuf.at[k + 2]], gb[nsl],
                                           gsems[nsl])
        sh[(KPB - 1) % 3].wait()
        return carry
    lax.fori_loop(0, NBB, block, 0)
    plsc.subcore_barrier()

    rows = pl.ds(s * RPS, RPS)
    pltpu.sync_copy(acc_sh.at[rows], out_h.at[c, rows])


BM = 1000


def _mm_body(x_ref, w_ref, o_ref):
    o_ref[...] = jnp.dot(x_ref[...], w_ref[...],
                         preferred_element_type=jnp.float32)


def _mm(x, W):
    M, Kin = x.shape
    Kout = W.shape[1]
    return pl.pallas_call(
        _mm_body,
        grid=(M // BM,),
        in_specs=[pl.BlockSpec((BM, Kin), lambda i: (i, 0)),
                  pl.BlockSpec((Kin, Kout), lambda i: (0, 0))],
        out_specs=pl.BlockSpec((BM, Kout), lambda i: (i, 0)),
        out_shape=jax.ShapeDtypeStruct((M, Kout), jnp.float32),
    )(x, W)


def _ep1_body(ap_ref, xw_ref, dv_ref, b_ref, w_ref, o_ref):
    dv = dv_ref[...]
    h = dv * (ap_ref[0] + ap_ref[1]) + (dv * dv) * xw_ref[...] + b_ref[...]
    h = jnp.maximum(h, 0.0)
    o_ref[...] = jnp.dot(h, w_ref[...], preferred_element_type=jnp.float32)


def _ep1(ap, xw, dv, b, W):
    M, Fin = xw.shape
    Kout = W.shape[1]
    return pl.pallas_call(
        _ep1_body,
        grid=(M // BM,),
        in_specs=[pl.BlockSpec((NC, BM, Fin), lambda i: (0, i, 0)),
                  pl.BlockSpec((BM, Fin), lambda i: (i, 0)),
                  pl.BlockSpec((BM, 1), lambda i: (i, 0)),
                  pl.BlockSpec((1, Fin), lambda i: (0, 0)),
                  pl.BlockSpec((Fin, Kout), lambda i: (0, 0))],
        out_specs=pl.BlockSpec((BM, Kout), lambda i: (i, 0)),
        out_shape=jax.ShapeDtypeStruct((M, Kout), jnp.float32),
    )(ap, xw, dv, b, W)


def _tail_body(ap_ref, xw_ref, dv_ref, b_ref, eps_ref,
               mu_ref, std_ref, z_ref):
    dv = dv_ref[...]
    h = dv * (ap_ref[0] + ap_ref[1]) + (dv * dv) * xw_ref[...] + b_ref[...]
    mu = h[:, :K]
    t = h[:, K:] - 5.0
    sp = jnp.maximum(t, 0.0) + jnp.log1p(jnp.exp(-jnp.abs(t)))
    mu_ref[...] = mu
    std_ref[...] = sp
    z_ref[...] = mu + eps_ref[...] * sp


def _tail(ap, xw, dv, b, eps):
    M, Fin = xw.shape
    out = jax.ShapeDtypeStruct((M, K), jnp.float32)
    return pl.pallas_call(
        _tail_body,
        grid=(M // BM,),
        in_specs=[pl.BlockSpec((NC, BM, Fin), lambda i: (0, i, 0)),
                  pl.BlockSpec((BM, Fin), lambda i: (i, 0)),
                  pl.BlockSpec((BM, 1), lambda i: (i, 0)),
                  pl.BlockSpec((1, Fin), lambda i: (0, 0)),
                  pl.BlockSpec((BM, K), lambda i: (i, 0))],
        out_specs=[pl.BlockSpec((BM, K), lambda i: (i, 0))] * 3,
        out_shape=(out, out, out),
    )(ap, xw, dv, b, eps)


def kernel(x, edge_index, edge_weight, W1, b1, W2, b2):
    row = edge_index[0]
    col = edge_index[1]
    row2 = row.reshape(NB3, KPB, CH)
    col2 = col.reshape(NB3, KPB, CH)
    ew2 = edge_weight.reshape(NB3, KPB, CH)

    norm2, dinv = _norm_kernel(row, edge_weight, col2, ew2)
    dv = dinv[:N].reshape(N, 1)

    xw1 = _mm(x, W1)
    ap1 = _spmm_kernel(xw1, row2, col2, norm2)
    xw2 = _ep1(ap1, xw1, dv, b1.reshape(1, -1), W2)

    ap2 = _spmm_kernel(xw2, row2, col2, norm2)
    eps = jax.random.normal(jax.random.key(42), (N, K), jnp.float32)
    mu, std, z = _tail(ap2, xw2, dv, b2.reshape(1, -1), eps)
    return (mu, std, z)


# speedup vs baseline: 1.1511x; 1.0330x over previous
import functools

import jax
import jax.numpy as jnp
from jax import lax
from jax.experimental import pallas as pl
from jax.experimental.pallas import tpu as pltpu
from jax.experimental.pallas import tpu_sc as plsc

N = 10000
E = 320000
K = 64
NC, NS, L = 2, 16, 16
NW = NC * NS
EPW = E // NW
N_PAD = NW * 320
NPW = N_PAD // NW
RPS = N_PAD // NS
CH = 80
KPB = 25
NB3 = E // (KPB * CH)
NBB = NB3 // NW
CH2 = 2000
F = 128

_mesh = plsc.VectorSubcoreMesh(
    core_axis_name="c", subcore_axis_name="s", num_cores=NC, num_subcores=NS)
_sc_params = pltpu.CompilerParams(needs_layout_passes=False)


def _rsqrt16(d):
    bits = plsc.bitcast(d, jnp.int32)
    y = plsc.bitcast(jnp.int32(0x5F3759DF) - (bits >> 1), jnp.float32)
    h = d * jnp.float32(-0.5)
    y = y * (jnp.float32(1.5) + h * y * y)
    y = y * (jnp.float32(1.5) + h * y * y)
    y = y * (jnp.float32(1.5) + h * y * y)
    return y


@functools.partial(
    pl.kernel,
    out_type=(jax.ShapeDtypeStruct((NB3, KPB, CH), jnp.float32),
              jax.ShapeDtypeStruct((N_PAD,), jnp.float32)),
    mesh=_mesh,
    scratch_types=[
        pltpu.VMEM_SHARED((N_PAD,), jnp.float32),
        pltpu.VMEM((N_PAD,), jnp.float32),
        pltpu.VMEM((N_PAD // NS,), jnp.float32),
        pltpu.VMEM((2, KPB, CH), jnp.int32),
        pltpu.VMEM((2, KPB, CH), jnp.float32),
        pltpu.VMEM((CH2,), jnp.int32),
        pltpu.VMEM((CH2,), jnp.float32),
        pltpu.VMEM((KPB, CH), jnp.float32),
        pltpu.SemaphoreType.DMA,
    ],
    compiler_params=_sc_params,
)
def _norm_kernel(row_h, ew_h, col2_h, ew2_h, norm2_h, dinv_h,
                 deg_sh, dloc, onesb, cbufD, ebufD, rbufN, ebufN, nbufN, semD):
    c = lax.axis_index("c")
    s = lax.axis_index("s")
    wid = s * NC + c

    spn = N_PAD // NS

    @plsc.parallel_loop(0, spn // L, unroll=4)
    def _(i):
        onesb[pl.ds(i * L, L)] = jnp.full((L,), 1.0, jnp.float32)
    pltpu.sync_copy(onesb, deg_sh.at[pl.ds(s * spn, spn)])
    plsc.subcore_barrier()

    bps = NB3 // NS
    cbs = [cbufD.at[r] for r in range(2)]
    ebs = [ebufD.at[r] for r in range(2)]

    def stage(blk, r):
        pltpu.sync_copy(col2_h.at[blk], cbs[r])
        pltpu.sync_copy(ew2_h.at[blk], ebs[r])

    def fire(r):
        return [pltpu.async_copy(ebs[r].at[k], deg_sh.at[cbs[r].at[k]],
                                 semD, add=True) for k in range(KPB)]

    stage(s * bps, 0)

    def deg_pair(p, carry):
        for r in range(2):
            b = p * 2 + r
            hs = fire(r)

            @pl.when(b + 1 < bps)
            def _():
                stage(s * bps + b + 1, (r + 1) % 2)
            for h in hs:
                h.wait()
        return carry
    lax.fori_loop(0, bps // 2, deg_pair, 0)
    plsc.subcore_barrier()

    pltpu.sync_copy(deg_sh, dloc)

    @plsc.parallel_loop(0, N_PAD // L, unroll=4)
    def _(i):
        dloc[pl.ds(i * L, L)] = _rsqrt16(dloc[pl.ds(i * L, L)])
    pltpu.sync_copy(dloc.at[pl.ds(wid * NPW, NPW)],
                    dinv_h.at[pl.ds(wid * NPW, NPW)])

    def norm_chunk(i, carry):
        off = wid * EPW + i * CH2
        pltpu.sync_copy(row_h.at[pl.ds(off, CH2)], rbufN)
        pltpu.sync_copy(ew_h.at[pl.ds(off, CH2)], ebufN)

        @plsc.parallel_loop(0, CH2 // L, unroll=4)
        def _(j):
            r = rbufN[pl.ds(j * L, L)]
            e = ebufN[pl.ds(j * L, L)]
            a = plsc.load_gather(dloc, [r])
            kk = j // (CH // L)
            jj = j % (CH // L)
            nbufN[kk, pl.ds(jj * L, L)] = a * e
        pltpu.sync_copy(nbufN, norm2_h.at[wid * NBB + i])
        return carry
    lax.fori_loop(0, EPW // CH2, norm_chunk, 0)


@functools.partial(
    pl.kernel,
    out_type=jax.ShapeDtypeStruct((NC, N_PAD, F), jnp.float32),
    mesh=_mesh,
    scratch_types=[
        pltpu.VMEM_SHARED((N_PAD, F), jnp.float32),
        pltpu.VMEM((3, CH, F), jnp.float32),
        pltpu.VMEM((KPB, CH), jnp.int32),
        pltpu.VMEM((KPB, CH), jnp.int32),
        pltpu.VMEM((KPB, CH), jnp.float32),
        pltpu.SemaphoreType.DMA,
        pltpu.SemaphoreType.DMA,
        pltpu.SemaphoreType.DMA,
        pltpu.SemaphoreType.DMA,
        pltpu.SemaphoreType.DMA,
        pltpu.SemaphoreType.DMA,
    ],
    compiler_params=_sc_params,
)
def _spmm_kernel(y_h, r2_h, c2_h, n2_h, out_h,
                 acc_sh, gbuf, rbuf, cbuf, nbuf, g0, g1, g2, s0, s1, s2):
    gsems = [g0, g1, g2]
    ssems = [s0, s1, s2]
    c = lax.axis_index("c")
    s = lax.axis_index("s")
    wid = s * NC + c
    gb = [gbuf.at[r] for r in range(3)]

    z = gb[0]

    @plsc.parallel_loop(0, CH, unroll=2)
    def _(i):
        for k in range(F // L):
            z[i, pl.ds(k * L, L)] = jnp.zeros((L,), jnp.float32)
    for t in range(RPS // CH):
        pltpu.sync_copy(z, acc_sh.at[pl.ds(s * RPS + t * CH, CH)])
    plsc.subcore_barrier()

    def scale_block(k, sl):
        g = gb[sl]
        kk = jnp.full((L,), k, jnp.int32)

        @plsc.parallel_loop(0, CH, unroll=4)
        def _(jj):
            nv = plsc.load_gather(
                nbuf, [kk, jnp.full((L,), 0, jnp.int32) + jj])
            for t in range(F // L):
                g[jj, pl.ds(t * L, L)] = g[jj, pl.ds(t * L, L)] * nv

    def block(b, carry):
        blk = wid * NBB + b
        pltpu.sync_copy(r2_h.at[blk], rbuf)
        pltpu.sync_copy(c2_h.at[blk], cbuf)
        pltpu.sync_copy(n2_h.at[blk], nbuf)
        gh = [None] * 3
        sh = [None] * 3
        gh[0] = pltpu.async_copy(y_h.at[rbuf.at[0]], gb[0], gsems[0])
        gh[1] = pltpu.async_copy(y_h.at[rbuf.at[1]], gb[1], gsems[1])
        for k in range(KPB):
            sl = k % 3
            gh[sl].wait()
            scale_block(k, sl)
            sh[sl] = pltpu.async_copy(gb[sl], acc_sh.at[cbuf.at[k]],
                                      ssems[sl], add=True)
            if k >= 1:
                sh[(k - 1) % 3].wait()
            if k + 2 < KPB:
                nsl = (k + 2) % 3
                gh[nsl] = pltpu.async_copy(y_h.at[rbuf.at[k + 2]], gb[nsl],
                                           gsems[nsl])
        sh[(KPB - 1) % 3].wait()
        return carry
    lax.fori_loop(0, NBB, block, 0)
    plsc.subcore_barrier()

    rows = pl.ds(s * RPS, RPS)
    pltpu.sync_copy(acc_sh.at[rows], out_h.at[c, rows])


BM = 1000


def _mm_body(x_ref, w_ref, o_ref):
    o_ref[...] = jnp.dot(x_ref[...], w_ref[...],
                         preferred_element_type=jnp.float32)


def _mm(x, W):
    M, Kin = x.shape
    Kout = W.shape[1]
    return pl.pallas_call(
        _mm_body,
        grid=(M // BM,),
        in_specs=[pl.BlockSpec((BM, Kin), lambda i: (i, 0)),
                  pl.BlockSpec((Kin, Kout), lambda i: (0, 0))],
        out_specs=pl.BlockSpec((BM, Kout), lambda i: (i, 0)),
        out_shape=jax.ShapeDtypeStruct((M, Kout), jnp.float32),
    )(x, W)


def _ep1_body(ap_ref, xw_ref, dv_ref, b_ref, w_ref, o_ref):
    dv = dv_ref[...]
    h = dv * (ap_ref[0] + ap_ref[1]) + (dv * dv) * xw_ref[...] + b_ref[...]
    h = jnp.maximum(h, 0.0)
    o_ref[...] = jnp.dot(h, w_ref[...], preferred_element_type=jnp.float32)


def _ep1(ap, xw, dv, b, W):
    M, Fin = xw.shape
    Kout = W.shape[1]
    return pl.pallas_call(
        _ep1_body,
        grid=(M // BM,),
        in_specs=[pl.BlockSpec((NC, BM, Fin), lambda i: (0, i, 0)),
                  pl.BlockSpec((BM, Fin), lambda i: (i, 0)),
                  pl.BlockSpec((BM, 1), lambda i: (i, 0)),
                  pl.BlockSpec((1, Fin), lambda i: (0, 0)),
                  pl.BlockSpec((Fin, Kout), lambda i: (0, 0))],
        out_specs=pl.BlockSpec((BM, Kout), lambda i: (i, 0)),
        out_shape=jax.ShapeDtypeStruct((M, Kout), jnp.float32),
    )(ap, xw, dv, b, W)


def _tail_body(ap_ref, xw_ref, dv_ref, b_ref, eps_ref,
               mu_ref, std_ref, z_ref):
    dv = dv_ref[...]
    h = dv * (ap_ref[0] + ap_ref[1]) + (dv * dv) * xw_ref[...] + b_ref[...]
    mu = h[:, :K]
    t = h[:, K:] - 5.0
    sp = jnp.maximum(t, 0.0) + jnp.log1p(jnp.exp(-jnp.abs(t)))
    mu_ref[...] = mu
    std_ref[...] = sp
    z_ref[...] = mu + eps_ref[...] * sp


def _tail(ap, xw, dv, b, eps):
    M, Fin = xw.shape
    out = jax.ShapeDtypeStruct((M, K), jnp.float32)
    return pl.pallas_call(
        _tail_body,
        grid=(M // BM,),
        in_specs=[pl.BlockSpec((NC, BM, Fin), lambda i: (0, i, 0)),
                  pl.BlockSpec((BM, Fin), lambda i: (i, 0)),
                  pl.BlockSpec((BM, 1), lambda i: (i, 0)),
                  pl.BlockSpec((1, Fin), lambda i: (0, 0)),
                  pl.BlockSpec((BM, K), lambda i: (i, 0))],
        out_specs=[pl.BlockSpec((BM, K), lambda i: (i, 0))] * 3,
        out_shape=(out, out, out),
    )(ap, xw, dv, b, eps)


def kernel(x, edge_index, edge_weight, W1, b1, W2, b2):
    row = edge_index[0]
    col = edge_index[1]
    row2 = row.reshape(NB3, KPB, CH)
    col2 = col.reshape(NB3, KPB, CH)
    ew2 = edge_weight.reshape(NB3, KPB, CH)

    norm2, dinv = _norm_kernel(row, edge_weight, col2, ew2)
    dv = dinv[:N].reshape(N, 1)

    xw1 = _mm(x, W1)
    ap1 = _spmm_kernel(xw1, row2, col2, norm2)
    xw2 = _ep1(ap1, xw1, dv, b1.reshape(1, -1), W2)

    ap2 = _spmm_kernel(xw2, row2, col2, norm2)
    eps = jax.random.normal(jax.random.key(42), (N, K), jnp.float32)
    mu, std, z = _tail(ap2, xw2, dv, b2.reshape(1, -1), eps)
    return (mu, std, z)
